# baseline (speedup 1.0000x reference)
import functools
import math

import jax
import jax.numpy as jnp
from jax import lax
from jax.experimental import pallas as pl
from jax.experimental.pallas import tpu as pltpu
from jax.experimental.pallas import tpu_sc as plsc

EMB_DIM = 64
SCALE = math.sqrt(EMB_DIM)
LANES = 16
CHUNK = 128


def _make_sc_gather(n_tokens: int, vocab: int, d: int):
  info = plsc.get_sparse_core_info()
  nw = info.num_cores * info.num_subcores
  assert n_tokens % (nw * CHUNK) == 0
  per_w = n_tokens // nw
  n_chunks = per_w // CHUNK

  mesh = plsc.VectorSubcoreMesh(core_axis_name="c", subcore_axis_name="s")

  @functools.partial(
      pl.kernel,
      mesh=mesh,
      out_type=jax.ShapeDtypeStruct((n_tokens, d), jnp.float32),
      scratch_types=[
          pltpu.VMEM((n_chunks, CHUNK), jnp.int32),
          pltpu.VMEM((CHUNK, d), jnp.float32),
          pltpu.SemaphoreType.DMA,
      ],
      compiler_params=pltpu.CompilerParams(use_tc_tiling_on_sc=False),
  )
  def gather_kernel(idx_hbm, table_hbm, out_hbm, idx_v, rows_v, sem):
    wid = lax.axis_index("s") * info.num_cores + lax.axis_index("c")
    base = wid * per_w
    pltpu.sync_copy(idx_hbm.at[wid], idx_v)

    def chunk_body(j, carry):
      pltpu.async_copy(table_hbm.at[idx_v.at[j]], rows_v, sem).wait()

      def scale_body(r, c):
        for v in range(d // LANES):
          sl = pl.ds(v * LANES, LANES)
          rows_v[r, sl] = rows_v[r, sl] * SCALE
        return c

      lax.fori_loop(0, CHUNK, scale_body, 0, unroll=2)
      pltpu.sync_copy(rows_v, out_hbm.at[pl.ds(base + j * CHUNK, CHUNK)])
      return carry

    lax.fori_loop(0, n_chunks, chunk_body, 0)

  return gather_kernel


@jax.jit
def kernel(tokens, table):
  b, s = tokens.shape
  vocab, d = table.shape
  n = b * s
  info = plsc.get_sparse_core_info()
  nw = info.num_cores * info.num_subcores
  idx = tokens.reshape(nw, n // (nw * CHUNK), CHUNK)
  out = _make_sc_gather(n, vocab, d)(idx, table)
  return out.reshape(b, s, d)


# speedup vs baseline: 1.1606x; 1.1606x over previous
import functools
import math

import jax
import jax.numpy as jnp
from jax import lax
from jax.experimental import pallas as pl
from jax.experimental.pallas import tpu as pltpu
from jax.experimental.pallas import tpu_sc as plsc

EMB_DIM = 64
SCALE = math.sqrt(EMB_DIM)
LANES = 16
CHUNK = 256
NBUF = 4


def _make_sc_gather(n_tokens: int, vocab: int, d: int):
  info = plsc.get_sparse_core_info()
  nw = info.num_cores * info.num_subcores
  assert n_tokens % (nw * CHUNK) == 0
  per_w = n_tokens // nw
  n_chunks = per_w // CHUNK
  assert n_chunks % NBUF == 0

  mesh = plsc.VectorSubcoreMesh(core_axis_name="c", subcore_axis_name="s")

  @functools.partial(
      pl.kernel,
      mesh=mesh,
      out_type=jax.ShapeDtypeStruct((n_tokens, d), jnp.float32),
      scratch_types=[
          pltpu.VMEM((n_chunks, CHUNK), jnp.int32),
      ]
      + [pltpu.VMEM((CHUNK, d), jnp.float32)] * NBUF
      + [pltpu.SemaphoreType.DMA] * (2 * NBUF),
      compiler_params=pltpu.CompilerParams(use_tc_tiling_on_sc=False),
  )
  def gather_kernel(idx_hbm, table_hbm, out_hbm, idx_v, *bufs_and_sems):
    rows = bufs_and_sems[:NBUF]
    gin = bufs_and_sems[NBUF : 2 * NBUF]
    gout = bufs_and_sems[2 * NBUF :]
    wid = lax.axis_index("s") * info.num_cores + lax.axis_index("c")
    base = wid * per_w
    pltpu.sync_copy(idx_hbm.at[wid], idx_v)

    def start_gather(j, b):
      pltpu.async_copy(table_hbm.at[idx_v.at[j]], rows[b], gin[b])

    def wait_gather(b):
      pltpu.make_async_copy(
          table_hbm.at[pl.ds(0, CHUNK)], rows[b], gin[b]
      ).wait()

    def start_out(j, b):
      pltpu.async_copy(rows[b], out_hbm.at[pl.ds(base + j * CHUNK, CHUNK)], gout[b])

    def wait_out(b):
      pltpu.make_async_copy(
          rows[b], out_hbm.at[pl.ds(base, CHUNK)], gout[b]
      ).wait()

    for c in range(NBUF - 1):
      start_gather(c, c)

    def body(i, carry):
      for b in range(NBUF):
        j = i * NBUF + b
        bn = (b + NBUF - 1) % NBUF
        if b == 0:

          @pl.when(j + NBUF - 1 < n_chunks)
          def _():
            @pl.when(j >= 1)
            def _():
              wait_out(bn)

            start_gather(j + NBUF - 1, bn)
        else:

          @pl.when(j + NBUF - 1 < n_chunks)
          def _():
            wait_out(bn)
            start_gather(j + NBUF - 1, bn)

        wait_gather(b)

        def scale_body(r, c2):
          for v in range(d // LANES):
            sl = pl.ds(v * LANES, LANES)
            rows[b][r, sl] = rows[b][r, sl] * SCALE
          return c2

        lax.fori_loop(0, CHUNK, scale_body, 0, unroll=4)
        start_out(j, b)
      return carry

    lax.fori_loop(0, n_chunks // NBUF, body, 0)
    for b in range(NBUF):
      wait_out(b)

  return gather_kernel


@jax.jit
def kernel(tokens, table):
  b, s = tokens.shape
  vocab, d = table.shape
  n = b * s
  info = plsc.get_sparse_core_info()
  nw = info.num_cores * info.num_subcores
  idx = tokens.reshape(nw, n // (nw * CHUNK), CHUNK)
  out = _make_sc_gather(n, vocab, d)(idx, table)
  return out.reshape(b, s, d)


# speedup vs baseline: 1.4168x; 1.2208x over previous
import functools
import math

import jax
import jax.numpy as jnp
from jax import lax
from jax.experimental import pallas as pl
from jax.experimental.pallas import tpu as pltpu
from jax.experimental.pallas import tpu_sc as plsc

EMB_DIM = 64
SCALE = math.sqrt(EMB_DIM)
LANES = 16
CHUNK = 128
NBUF = 4


def _make_sc_gather(n_tokens: int, vocab: int, d: int, dpad: int):
  info = plsc.get_sparse_core_info()
  nw = info.num_cores * info.num_subcores
  assert n_tokens % (nw * CHUNK) == 0
  per_w = n_tokens // nw
  n_chunks = per_w // CHUNK
  assert n_chunks % NBUF == 0
  idx_rows = per_w // 128

  mesh = plsc.VectorSubcoreMesh(core_axis_name="c", subcore_axis_name="s")

  @functools.partial(
      pl.kernel,
      mesh=mesh,
      out_type=jax.ShapeDtypeStruct((n_tokens, dpad), jnp.float32),
      scratch_types=[
          pltpu.VMEM((idx_rows, 128), jnp.int32),
      ]
      + [pltpu.VMEM((CHUNK, dpad), jnp.float32)] * NBUF
      + [pltpu.SemaphoreType.DMA] * (2 * NBUF),
  )
  def gather_kernel(idx_hbm, table_hbm, out_hbm, idx_v, *bufs_and_sems):
    rows = bufs_and_sems[:NBUF]
    gin = bufs_and_sems[NBUF : 2 * NBUF]
    gout = bufs_and_sems[2 * NBUF :]
    wid = lax.axis_index("s") * info.num_cores + lax.axis_index("c")
    base = wid * per_w
    pltpu.sync_copy(idx_hbm.at[pl.ds(wid * idx_rows, idx_rows)], idx_v)

    def start_gather(j, b):
      pltpu.async_copy(table_hbm.at[idx_v.at[j]], rows[b], gin[b])

    def wait_gather(b):
      pltpu.make_async_copy(
          table_hbm.at[pl.ds(0, CHUNK)], rows[b], gin[b]
      ).wait()

    def start_out(j, b):
      pltpu.async_copy(
          rows[b], out_hbm.at[pl.ds(base + j * CHUNK, CHUNK)], gout[b]
      )

    def wait_out(b):
      pltpu.make_async_copy(
          rows[b], out_hbm.at[pl.ds(base, CHUNK)], gout[b]
      ).wait()

    for c in range(NBUF - 1):
      start_gather(c, c)

    def body(i, carry):
      for b in range(NBUF):
        j = i * NBUF + b
        bn = (b + NBUF - 1) % NBUF
        if b == 0:

          @pl.when(j + NBUF - 1 < n_chunks)
          def _():
            @pl.when(j >= 1)
            def _():
              wait_out(bn)

            start_gather(j + NBUF - 1, bn)
        else:

          @pl.when(j + NBUF - 1 < n_chunks)
          def _():
            wait_out(bn)
            start_gather(j + NBUF - 1, bn)

        wait_gather(b)

        def scale_body(r, c2):
          for v in range(d // LANES):
            sl = pl.ds(v * LANES, LANES)
            rows[b][r, sl] = rows[b][r, sl] * SCALE
          return c2

        lax.fori_loop(0, CHUNK, scale_body, 0, unroll=4)
        start_out(j, b)
      return carry

    lax.fori_loop(0, n_chunks // NBUF, body, 0)
    for b in range(NBUF):
      wait_out(b)

  return gather_kernel


@jax.jit
def kernel(tokens, table):
  b, s = tokens.shape
  vocab, d = table.shape
  dpad = 2 * d
  n = b * s
  idx = tokens.reshape(n // 128, 128)
  table_p = jnp.pad(table, ((0, 0), (0, dpad - d)))
  out = _make_sc_gather(n, vocab, d, dpad)(idx, table_p)
  return out[:, :d].reshape(b, s, d)
